# baseline (speedup 1.0000x reference)
import functools

import jax
import jax.numpy as jnp
from jax import lax
from jax.experimental import pallas as pl
from jax.experimental.pallas import tpu as pltpu
from jax.experimental.pallas import tpu_sc as plsc

R = 64
N = 8192
K = 256
L = 16
NV = N // L
NC = 2
NS = 16
ROWS_PER_W = R // (NC * NS)


def _sc_body(x_hbm, o_hbm, x_v, key_v, cand0_v, cand1_v, hist_v, totals_v,
             suffix_v, out_v):
    MIN32 = jnp.int32(-2147483648)
    lane = jnp.arange(L, dtype=jnp.int32)
    ones = jnp.ones((L,), jnp.int32)
    zeros = jnp.zeros((L,), jnp.int32)
    kk = jnp.int32(K)

    wid = lax.axis_index("s") * NC + lax.axis_index("c")

    def clear_hist():
        def clr(i, _):
            hist_v[pl.ds(i * L, L)] = zeros
            return 0
        lax.fori_loop(0, (L * 256) // L, clr, 0)

    def select_bucket(k_rem):
        for j in range(16):
            acc = zeros
            for l in range(L):
                acc = acc + hist_v[pl.ds(l * 256 + j * L, L)]
            totals_v[pl.ds(j * L, L)] = acc
        chunk_sums = []
        for j in range(16):
            tj = totals_v[pl.ds(j * L, L)]
            chunk_sums.append(jnp.sum(tj))
        above = jnp.int32(0)
        pcacc = zeros
        for j in range(15, -1, -1):
            tj = totals_v[pl.ds(j * L, L)]
            rc = lax.rev(plsc.cumsum(lax.rev(tj, (0,))), (0,))
            suf = rc + above
            suffix_v[pl.ds(j * L, L)] = suf
            pcacc = pcacc + (suf >= k_rem).astype(jnp.int32)
            above = above + chunk_sums[j]
        b_star = jnp.sum(pcacc) - jnp.int32(1)
        tacc = zeros
        sacc = zeros
        for j in range(16):
            gidx = lane + jnp.int32(j * L)
            sel = (gidx == b_star).astype(jnp.int32)
            tacc = tacc + sel * totals_v[pl.ds(j * L, L)]
            sacc = sacc + sel * suffix_v[pl.ds(j * L, L)]
        tot_b = jnp.max(tacc)
        suf_b = jnp.max(sacc)
        c_above = suf_b - tot_b
        return b_star, k_rem - c_above

    def do_row(row, _):
        pltpu.sync_copy(x_hbm.at[row], x_v)

        clear_hist()

        def p1(i, _):
            xv = x_v[pl.ds(i * L, L)]
            b = lax.bitcast_convert_type(xv, jnp.int32)
            sk = jnp.where(b >= 0, b, MIN32 - b)
            key_v[pl.ds(i * L, L)] = sk
            bucket = (sk >> 24) + jnp.int32(128)
            plsc.addupdate_scatter(hist_v, [lane * 256 + bucket], ones)
            return 0
        lax.fori_loop(0, NV, p1, 0)
        b0, k2 = select_bucket(kk)

        clear_hist()

        def p2(i, off):
            sk = key_v[pl.ds(i * L, L)]
            m = ((sk >> 24) + jnp.int32(128)) == b0
            plsc.store_scatter(cand0_v, [lane * NV + off], sk, mask=m)
            b1 = (sk >> 16) & jnp.int32(0xFF)
            plsc.addupdate_scatter(hist_v, [lane * 256 + b1], ones, mask=m)
            return off + m.astype(jnp.int32)
        off0 = lax.fori_loop(0, NV, p2, zeros)
        b1s, k3 = select_bucket(k2)

        clear_hist()
        max0 = jnp.max(off0)

        def p3(i, off):
            valid = i < off0
            sk = plsc.load_gather(cand0_v, [lane * NV + i], mask=valid)
            m = valid & (((sk >> 16) & jnp.int32(0xFF)) == b1s)
            plsc.store_scatter(cand1_v, [lane * NV + off], sk, mask=m)
            b2 = (sk >> 8) & jnp.int32(0xFF)
            plsc.addupdate_scatter(hist_v, [lane * 256 + b2], ones, mask=m)
            return off + m.astype(jnp.int32)
        off1 = lax.fori_loop(0, max0, p3, zeros)
        b2s, k4 = select_bucket(k3)

        clear_hist()
        max1 = jnp.max(off1)

        def p4(i, _):
            valid = i < off1
            sk = plsc.load_gather(cand1_v, [lane * NV + i], mask=valid)
            m = valid & (((sk >> 8) & jnp.int32(0xFF)) == b2s)
            b3 = sk & jnp.int32(0xFF)
            plsc.addupdate_scatter(hist_v, [lane * 256 + b3], ones, mask=m)
            return 0
        lax.fori_loop(0, max1, p4, 0)
        b3s, need = select_bucket(k4)

        t = (lax.shift_left(b0 - jnp.int32(128), jnp.int32(24))
             | lax.shift_left(b1s, jnp.int32(16))
             | lax.shift_left(b2s, jnp.int32(8)) | b3s)

        def pout(i, cnt):
            sk = key_v[pl.ds(i * L, L)]
            xv = x_v[pl.ds(i * L, L)]
            ge = sk >= t
            out_v[pl.ds(i * L, L)] = jnp.where(ge, xv, jnp.float32(0.0))
            return cnt + ge.astype(jnp.int32)
        cntge = lax.fori_loop(0, NV, pout, zeros)
        total_ge = jnp.sum(cntge)

        @pl.when(total_ge != kk)
        def _fixup():
            def pfix(i, c):
                sk = key_v[pl.ds(i * L, L)]
                xv = x_v[pl.ds(i * L, L)]
                eq = sk == t
                eq_i = eq.astype(jnp.int32)
                pre = plsc.cumsum(eq_i) + c
                keep = (sk > t) | (eq & (pre <= need))
                out_v[pl.ds(i * L, L)] = jnp.where(keep, xv, jnp.float32(0.0))
                return c + jnp.sum(eq_i)
            lax.fori_loop(0, NV, pfix, jnp.int32(0))

        pltpu.sync_copy(out_v, o_hbm.at[row])
        return 0

    lax.fori_loop(wid * ROWS_PER_W, (wid + 1) * ROWS_PER_W, do_row, 0)


def kernel(x, sparse_dim):
    del sparse_dim
    mesh = plsc.VectorSubcoreMesh(core_axis_name="c", subcore_axis_name="s",
                                  num_cores=NC, num_subcores=NS)
    f = pl.kernel(
        _sc_body,
        out_type=jax.ShapeDtypeStruct((R, N), jnp.float32),
        mesh=mesh,
        scratch_types=[
            pltpu.VMEM((N,), jnp.float32),
            pltpu.VMEM((N,), jnp.int32),
            pltpu.VMEM((N,), jnp.int32),
            pltpu.VMEM((N,), jnp.int32),
            pltpu.VMEM((L * 256,), jnp.int32),
            pltpu.VMEM((256,), jnp.int32),
            pltpu.VMEM((256,), jnp.int32),
            pltpu.VMEM((N,), jnp.float32),
        ],
        compiler_params=pltpu.CompilerParams(use_tc_tiling_on_sc=False,
                                             needs_layout_passes=False),
    )
    return f(x)


# speedup vs baseline: 1.7674x; 1.7674x over previous
import functools

import jax
import jax.numpy as jnp
from jax import lax
from jax.experimental import pallas as pl
from jax.experimental.pallas import tpu as pltpu
from jax.experimental.pallas import tpu_sc as plsc

R = 64
N = 8192
K = 256
L = 16
NV = N // L
NC = 2
NS = 16
ROWS_PER_W = R // (NC * NS)


def _sc_body(x_hbm, o_hbm, x_v, key_v, cand0_v, cand1_v, hist_v, totals_v,
             out_v):
    MIN32 = jnp.int32(-2147483648)
    lane = jnp.arange(L, dtype=jnp.int32)
    ones = jnp.ones((L,), jnp.int32)
    zeros = jnp.zeros((L,), jnp.int32)
    zidx = jnp.zeros((L,), jnp.int32)
    kk = jnp.int32(K)

    wid = lax.axis_index("s") * NC + lax.axis_index("c")

    def clear_hist():
        @plsc.parallel_loop(0, 256, unroll=8)
        def _clr(i):
            hist_v[pl.ds(i * L, L)] = zeros

    def splat0(v):
        return v.at[zidx].get(mode="promise_in_bounds")

    def select_bucket(k_rem):
        @plsc.parallel_loop(0, 16)
        def _tot(j):
            acc = hist_v[pl.ds(j * L, L)]
            for l in range(1, L):
                acc = acc + hist_v[pl.ds(l * 256 + j * L, L)]
            totals_v[pl.ds(j * L, L)] = acc
        above = zeros
        pcacc = zeros
        sufsel = zeros
        totsel = zeros
        b_hi = zeros
        taken = zeros
        for j in range(15, -1, -1):
            tj = totals_v[pl.ds(j * L, L)]
            rc = lax.rev(plsc.cumsum(lax.rev(tj, (0,))), (0,))
            suf = rc + above
            hit = suf >= k_rem
            hit_i = hit.astype(jnp.int32)
            pcacc = pcacc + hit_i
            above = above + splat0(rc)
            anyhit = splat0(hit_i)
            upd = (anyhit * (jnp.int32(1) - taken)) > 0
            sufsel = jnp.where(upd & hit, suf, sufsel)
            totsel = jnp.where(upd & hit, tj, totsel)
            b_hi = jnp.where(upd & hit, lane + jnp.int32(j * L), b_hi)
            taken = jnp.where(upd, ones, taken)
        b_star = jnp.sum(pcacc) - jnp.int32(1)
        sel = (b_hi == b_star).astype(jnp.int32)
        tot_b = jnp.max(sel * totsel)
        suf_b = jnp.max(sel * sufsel)
        c_above = suf_b - tot_b
        return b_star, k_rem - c_above

    def do_row(row, _):
        pltpu.sync_copy(x_hbm.at[row], x_v)

        clear_hist()

        @plsc.parallel_loop(0, NV, unroll=4)
        def p1(i):
            xv = x_v[pl.ds(i * L, L)]
            b = lax.bitcast_convert_type(xv, jnp.int32)
            sk = jnp.where(b >= 0, b, MIN32 - b)
            key_v[pl.ds(i * L, L)] = sk
            bucket = (sk >> 24) + jnp.int32(128)
            plsc.addupdate_scatter(hist_v, [lane * 256 + bucket], ones)
        b0, k2 = select_bucket(kk)

        clear_hist()

        @plsc.parallel_loop(0, NV, unroll=4, carry=zeros)
        def p2(i, off):
            sk = key_v[pl.ds(i * L, L)]
            m = ((sk >> 24) + jnp.int32(128)) == b0
            plsc.store_scatter(cand0_v, [lane * NV + off], sk, mask=m)
            b1 = (sk >> 16) & jnp.int32(0xFF)
            plsc.addupdate_scatter(hist_v, [lane * 256 + b1], ones, mask=m)
            return off + m.astype(jnp.int32)
        off0 = p2
        b1s, k3 = select_bucket(k2)

        clear_hist()
        max0 = jnp.max(off0)

        @plsc.parallel_loop(0, max0, carry=zeros)
        def p3(i, off):
            valid = i < off0
            sk = plsc.load_gather(cand0_v, [lane * NV + i], mask=valid)
            m = valid & (((sk >> 16) & jnp.int32(0xFF)) == b1s)
            plsc.store_scatter(cand1_v, [lane * NV + off], sk, mask=m)
            b2 = (sk >> 8) & jnp.int32(0xFF)
            plsc.addupdate_scatter(hist_v, [lane * 256 + b2], ones, mask=m)
            return off + m.astype(jnp.int32)
        off1 = p3
        b2s, k4 = select_bucket(k3)

        clear_hist()
        max1 = jnp.max(off1)

        @plsc.parallel_loop(0, max1)
        def p4(i):
            valid = i < off1
            sk = plsc.load_gather(cand1_v, [lane * NV + i], mask=valid)
            m = valid & (((sk >> 8) & jnp.int32(0xFF)) == b2s)
            b3 = sk & jnp.int32(0xFF)
            plsc.addupdate_scatter(hist_v, [lane * 256 + b3], ones, mask=m)
        b3s, need = select_bucket(k4)

        t = (lax.shift_left(b0 - jnp.int32(128), jnp.int32(24))
             | lax.shift_left(b1s, jnp.int32(16))
             | lax.shift_left(b2s, jnp.int32(8)) | b3s)

        @plsc.parallel_loop(0, NV, unroll=4, carry=zeros)
        def pout(i, cnt):
            sk = key_v[pl.ds(i * L, L)]
            xv = x_v[pl.ds(i * L, L)]
            ge = sk >= t
            out_v[pl.ds(i * L, L)] = jnp.where(ge, xv, jnp.float32(0.0))
            return cnt + ge.astype(jnp.int32)
        total_ge = jnp.sum(pout)

        @pl.when(total_ge != kk)
        def _fixup():
            def pfix(i, c):
                sk = key_v[pl.ds(i * L, L)]
                xv = x_v[pl.ds(i * L, L)]
                eq = sk == t
                eq_i = eq.astype(jnp.int32)
                pre = plsc.cumsum(eq_i) + c
                keep = (sk > t) | (eq & (pre <= need))
                out_v[pl.ds(i * L, L)] = jnp.where(keep, xv, jnp.float32(0.0))
                return c + jnp.sum(eq_i)
            lax.fori_loop(0, NV, pfix, jnp.int32(0))

        pltpu.sync_copy(out_v, o_hbm.at[row])
        return 0

    lax.fori_loop(wid * ROWS_PER_W, (wid + 1) * ROWS_PER_W, do_row, 0)


def kernel(x, sparse_dim):
    del sparse_dim
    mesh = plsc.VectorSubcoreMesh(core_axis_name="c", subcore_axis_name="s",
                                  num_cores=NC, num_subcores=NS)
    f = pl.kernel(
        _sc_body,
        out_type=jax.ShapeDtypeStruct((R, N), jnp.float32),
        mesh=mesh,
        scratch_types=[
            pltpu.VMEM((N,), jnp.float32),
            pltpu.VMEM((N,), jnp.int32),
            pltpu.VMEM((N,), jnp.int32),
            pltpu.VMEM((N,), jnp.int32),
            pltpu.VMEM((L * 256,), jnp.int32),
            pltpu.VMEM((256,), jnp.int32),
            pltpu.VMEM((N,), jnp.float32),
        ],
        compiler_params=pltpu.CompilerParams(use_tc_tiling_on_sc=False,
                                             needs_layout_passes=False),
    )
    return f(x)
